# baseline (speedup 1.0000x reference)
import functools

import jax
import jax.numpy as jnp
from jax import lax
from jax.experimental import pallas as pl
from jax.experimental.pallas import tpu as pltpu
from jax.experimental.pallas import tpu_sc as plsc

N = 10000
E = 320000
ESEG = 160000
EP = 161792
D = 128

NC = 2
NS = 16
NW = NC * NS
EPW = ESEG // NW
C = 40
NCHUNK = EPW // C
NSUPER = (NCHUNK - 1) // 2
NP = 10240
ROWS_PER_TILE = NP // NS

_MESH = plsc.VectorSubcoreMesh(core_axis_name="c", subcore_axis_name="s")


def _stage_a(row, col, curv, x, off):
  @functools.partial(
      pl.kernel,
      mesh=_MESH,
      out_type=[
          jax.ShapeDtypeStruct((EP, D), jnp.float32),
          jax.ShapeDtypeStruct((EP,), jnp.float32),
      ],
      scratch_types=[
          pltpu.VMEM((2, C), jnp.int32),
          pltpu.VMEM((2, C), jnp.int32),
          pltpu.VMEM((2, C, D), jnp.float32),
          pltpu.VMEM((2, C), jnp.float32),
          pltpu.VMEM((2, C), jnp.float32),
          pltpu.VMEM((2, C), jnp.float32),
          pltpu.SemaphoreType.DMA,
          pltpu.SemaphoreType.DMA,
      ],
  )
  def a_kernel(row_hbm, col_hbm, curv_hbm, x_hbm, xg_out, dif_out,
               ridx_v, cidx_v, rowsg_v, cr_v, cc_v, dif_v, semA, semB):
    cid = lax.axis_index("c")
    sid = lax.axis_index("s")
    base = (cid * NS + sid) * EPW
    sems = (semA, semB)

    def fire(g, p):
      eb = base + g * C
      pltpu.sync_copy(row_hbm.at[pl.ds(off + eb, C)], ridx_v.at[p])
      pltpu.sync_copy(col_hbm.at[pl.ds(off + eb, C)], cidx_v.at[p])
      pltpu.async_copy(x_hbm.at[cidx_v.at[p]], rowsg_v.at[p], sems[p])
      pltpu.async_copy(curv_hbm.at[ridx_v.at[p]], cr_v.at[p], sems[p])
      pltpu.async_copy(curv_hbm.at[cidx_v.at[p]], cc_v.at[p], sems[p])

    def wait(p):
      pltpu.make_async_copy(x_hbm.at[cidx_v.at[p]], rowsg_v.at[p], sems[p]).wait()
      pltpu.make_async_copy(curv_hbm.at[ridx_v.at[p]], cr_v.at[p], sems[p]).wait()
      pltpu.make_async_copy(curv_hbm.at[ridx_v.at[p]], cc_v.at[p], sems[p]).wait()

    def emit(g, p):
      for o in (0, 16, C - 16):
        sl = pl.ds(o, 16)
        dif_v[p, sl] = jnp.abs(cr_v[p, sl] - cc_v[p, sl])
      eb = base + g * C
      pltpu.sync_copy(rowsg_v.at[p], xg_out.at[pl.ds(eb, C)])
      pltpu.sync_copy(dif_v.at[p], dif_out.at[pl.ds(eb, C)])

    fire(0, 0)

    def superstep(i, _):
      g0 = 2 * i
      fire(g0 + 1, 1)
      wait(0)
      emit(g0, 0)
      fire(g0 + 2, 0)
      wait(1)
      emit(g0 + 1, 1)
      return 0

    lax.fori_loop(0, NSUPER, superstep, 0)
    wait(0)
    emit(NCHUNK - 1, 0)

  return a_kernel(row, col, curv, x)


def _gate_body(xg_ref, dif_ref, wc_ref, bc_ref, out_ref):
  z = dif_ref[...][:, None] * wc_ref[...] + bc_ref[...]
  w = 1.0 / (1.0 + jnp.exp(-z))
  out_ref[...] = xg_ref[...] * w


def _stage_b(xg, dif, wc, bc):
  B = 2048
  grid = EP // B
  return pl.pallas_call(
      _gate_body,
      grid=(grid,),
      in_specs=[
          pl.BlockSpec((B, D), lambda i: (i, 0)),
          pl.BlockSpec((B,), lambda i: (i,)),
          pl.BlockSpec((1, D), lambda i: (0, 0)),
          pl.BlockSpec((1, D), lambda i: (0, 0)),
      ],
      out_specs=pl.BlockSpec((B, D), lambda i: (i, 0)),
      out_shape=jax.ShapeDtypeStruct((EP, D), jnp.float32),
  )(xg, dif, wc[None, :], bc[None, :])


def _stage_c(row, vals, off):
  @functools.partial(
      pl.kernel,
      mesh=_MESH,
      out_type=[
          jax.ShapeDtypeStruct((NC, NP, D), jnp.float32),
          jax.ShapeDtypeStruct((NC, NP), jnp.float32),
      ],
      scratch_types=[
          pltpu.VMEM((2, C), jnp.int32),
          pltpu.VMEM((2, C, D), jnp.float32),
          pltpu.VMEM((C,), jnp.float32),
          pltpu.VMEM((ROWS_PER_TILE,), jnp.float32),
          pltpu.VMEM_SHARED((NP, D), jnp.float32),
          pltpu.VMEM_SHARED((NP,), jnp.float32),
          pltpu.SemaphoreType.DMA,
          pltpu.SemaphoreType.DMA,
      ],
  )
  def c_kernel(row_hbm, vals_hbm, acc_out, cnt_out,
               ridx_v, valsb_v, ones_v, zcnt_v, acc_s, cnt_s, semA, semB):
    cid = lax.axis_index("c")
    sid = lax.axis_index("s")
    base = (cid * NS + sid) * EPW
    sems = (semA, semB)

    zero16 = jnp.zeros((16,), jnp.float32)
    one16 = jnp.ones((16,), jnp.float32)

    def zfill(i, _):
      for k in range(D // 16):
        valsb_v[0, i, pl.ds(k * 16, 16)] = zero16
      return 0
    lax.fori_loop(0, C, zfill, 0)

    def zcnt_fill(i, _):
      zcnt_v[pl.ds(i * 16, 16)] = zero16
      return 0
    lax.fori_loop(0, ROWS_PER_TILE // 16, zcnt_fill, 0)

    for o in (0, 16, C - 16):
      ones_v[pl.ds(o, 16)] = one16

    for j in range(ROWS_PER_TILE // C):
      pltpu.sync_copy(valsb_v.at[0], acc_s.at[pl.ds(sid * ROWS_PER_TILE + j * C, C)])
    pltpu.sync_copy(zcnt_v, cnt_s.at[pl.ds(sid * ROWS_PER_TILE, ROWS_PER_TILE)])
    plsc.subcore_barrier()

    def fire(g, p):
      eb = base + g * C
      pltpu.sync_copy(row_hbm.at[pl.ds(off + eb, C)], ridx_v.at[p])
      pltpu.async_copy(vals_hbm.at[pl.ds(eb, C)], valsb_v.at[p], sems[p])

    def wait(p):
      pltpu.make_async_copy(vals_hbm.at[pl.ds(0, C)], valsb_v.at[p], sems[p]).wait()

    def scatter(p):
      pltpu.sync_copy(valsb_v.at[p], acc_s.at[ridx_v.at[p]], add=True)
      pltpu.sync_copy(ones_v, cnt_s.at[ridx_v.at[p]], add=True)

    fire(0, 0)

    def superstep(i, _):
      g0 = 2 * i
      fire(g0 + 1, 1)
      wait(0)
      scatter(0)
      fire(g0 + 2, 0)
      wait(1)
      scatter(1)
      return 0

    lax.fori_loop(0, NSUPER, superstep, 0)
    wait(0)
    scatter(0)
    plsc.subcore_barrier()

    pltpu.sync_copy(acc_s.at[pl.ds(sid * ROWS_PER_TILE, ROWS_PER_TILE)],
                    acc_out.at[cid, pl.ds(sid * ROWS_PER_TILE, ROWS_PER_TILE)])
    pltpu.sync_copy(cnt_s.at[pl.ds(sid * ROWS_PER_TILE, ROWS_PER_TILE)],
                    cnt_out.at[cid, pl.ds(sid * ROWS_PER_TILE, ROWS_PER_TILE)])

  return c_kernel(row, vals)


def _tc_finish_body(acc_ref, cnt_ref, acc2_ref, cnt2_ref, wl_ref, bl_ref, out_ref):
  feat = acc_ref[0] + acc_ref[1] + acc2_ref[0] + acc2_ref[1]
  cnt = cnt_ref[0] + cnt_ref[1] + cnt2_ref[0] + cnt2_ref[1]
  inv = 1.0 / jnp.maximum(cnt, 1.0)
  mean = feat * inv[:, None]
  h = lax.dot_general(mean, wl_ref[...], (((1,), (1,)), ((), ())),
                      preferred_element_type=jnp.float32)
  h = h + bl_ref[...][None, :]
  out_ref[...] = 0.5 * h * (1.0 + lax.erf(h * (2.0 ** -0.5)))


def _tc_finish(acc, cnt, acc2, cnt2, W_lin, b_lin):
  return pl.pallas_call(
      _tc_finish_body,
      out_shape=jax.ShapeDtypeStruct((NP, D), jnp.float32),
  )(acc, cnt, acc2, cnt2, W_lin, b_lin)


@jax.jit
def kernel(x, edge_index, curvature, W_lin, b_lin, W_curv, b_curv):
  row = edge_index[0]
  col = edge_index[1]
  wc = W_curv[:, 0]
  xg0, d0 = _stage_a(row, col, curvature, x, 0)
  v0 = _stage_b(xg0, d0, wc, b_curv)
  xg1, d1 = _stage_a(row, col, curvature, x, ESEG)
  v1 = _stage_b(xg1, d1, wc, b_curv)
  acc0, cnt0 = _stage_c(row, v0, 0)
  acc1, cnt1 = _stage_c(row, v1, ESEG)
  return _tc_finish(acc0, cnt0, acc1, cnt1, W_lin, b_lin)[:N]


# speedup vs baseline: 1.2989x; 1.2989x over previous
import functools

import jax
import jax.numpy as jnp
from jax import lax
from jax.experimental import pallas as pl
from jax.experimental.pallas import tpu as pltpu
from jax.experimental.pallas import tpu_sc as plsc

N = 10000
E = 320000
D = 128

NC = 2
NS = 16
NW = NC * NS
C = 80
NP = 10240
ROWS_PER_TILE = NP // NS

SEGS = (
    dict(off=0, epw=6000, nchunk=75, ep=94 * 2048),
    dict(off=192000, epw=4000, nchunk=50, ep=63 * 2048),
)

_MESH = plsc.VectorSubcoreMesh(core_axis_name="c", subcore_axis_name="s")


def _stage_a(row, col, curv, x, off, epw, nchunk, ep):
  nsuper = (nchunk - 1) // 2 if nchunk % 2 else (nchunk - 2) // 2

  @functools.partial(
      pl.kernel,
      mesh=_MESH,
      out_type=[
          jax.ShapeDtypeStruct((ep, D), jnp.float32),
          jax.ShapeDtypeStruct((ep,), jnp.float32),
      ],
      scratch_types=[
          pltpu.VMEM((2, C), jnp.int32),
          pltpu.VMEM((2, C), jnp.int32),
          pltpu.VMEM((2, C, D), jnp.float32),
          pltpu.VMEM((2, C), jnp.float32),
          pltpu.VMEM((2, C), jnp.float32),
          pltpu.VMEM((2, C), jnp.float32),
          pltpu.SemaphoreType.DMA,
          pltpu.SemaphoreType.DMA,
      ],
  )
  def a_kernel(row_hbm, col_hbm, curv_hbm, x_hbm, xg_out, dif_out,
               ridx_v, cidx_v, rowsg_v, cr_v, cc_v, dif_v, semA, semB):
    cid = lax.axis_index("c")
    sid = lax.axis_index("s")
    base = (cid * NS + sid) * epw
    sems = (semA, semB)

    def fire(g, p):
      eb = base + g * C
      pltpu.sync_copy(row_hbm.at[pl.ds(off + eb, C)], ridx_v.at[p])
      pltpu.sync_copy(col_hbm.at[pl.ds(off + eb, C)], cidx_v.at[p])
      pltpu.async_copy(x_hbm.at[cidx_v.at[p]], rowsg_v.at[p], sems[p])
      pltpu.async_copy(curv_hbm.at[ridx_v.at[p]], cr_v.at[p], sems[p])
      pltpu.async_copy(curv_hbm.at[cidx_v.at[p]], cc_v.at[p], sems[p])

    def wait(p):
      pltpu.make_async_copy(x_hbm.at[cidx_v.at[p]], rowsg_v.at[p], sems[p]).wait()
      pltpu.make_async_copy(curv_hbm.at[ridx_v.at[p]], cr_v.at[p], sems[p]).wait()
      pltpu.make_async_copy(curv_hbm.at[ridx_v.at[p]], cc_v.at[p], sems[p]).wait()

    def emit(g, p):
      for o in range(0, C, 16):
        sl = pl.ds(o, 16)
        dif_v[p, sl] = jnp.abs(cr_v[p, sl] - cc_v[p, sl])
      eb = base + g * C
      pltpu.sync_copy(rowsg_v.at[p], xg_out.at[pl.ds(eb, C)])
      pltpu.sync_copy(dif_v.at[p], dif_out.at[pl.ds(eb, C)])

    fire(0, 0)

    def superstep(i, _):
      g0 = 2 * i
      fire(g0 + 1, 1)
      wait(0)
      emit(g0, 0)
      fire(g0 + 2, 0)
      wait(1)
      emit(g0 + 1, 1)
      return 0

    lax.fori_loop(0, nsuper, superstep, 0)
    if nchunk % 2:
      wait(0)
      emit(nchunk - 1, 0)
    else:
      fire(nchunk - 1, 1)
      wait(0)
      emit(nchunk - 2, 0)
      wait(1)
      emit(nchunk - 1, 1)

  return a_kernel(row, col, curv, x)


def _gate_body(xg_ref, dif_ref, wc_ref, bc_ref, out_ref):
  z = dif_ref[...][:, None] * wc_ref[...] + bc_ref[...]
  w = 1.0 / (1.0 + jnp.exp(-z))
  out_ref[...] = xg_ref[...] * w


def _stage_b(xg, dif, wc, bc, ep):
  B = 2048
  grid = ep // B
  return pl.pallas_call(
      _gate_body,
      grid=(grid,),
      in_specs=[
          pl.BlockSpec((B, D), lambda i: (i, 0)),
          pl.BlockSpec((B,), lambda i: (i,)),
          pl.BlockSpec((1, D), lambda i: (0, 0)),
          pl.BlockSpec((1, D), lambda i: (0, 0)),
      ],
      out_specs=pl.BlockSpec((B, D), lambda i: (i, 0)),
      out_shape=jax.ShapeDtypeStruct((ep, D), jnp.float32),
  )(xg, dif, wc[None, :], bc[None, :])


def _stage_c(row, vals, off, epw, nchunk):
  nsuper = (nchunk - 1) // 2 if nchunk % 2 else (nchunk - 2) // 2

  @functools.partial(
      pl.kernel,
      mesh=_MESH,
      out_type=[
          jax.ShapeDtypeStruct((NC, NP, D), jnp.float32),
          jax.ShapeDtypeStruct((NC, NP), jnp.float32),
      ],
      scratch_types=[
          pltpu.VMEM((2, C), jnp.int32),
          pltpu.VMEM((2, C, D), jnp.float32),
          pltpu.VMEM((C,), jnp.float32),
          pltpu.VMEM((ROWS_PER_TILE,), jnp.float32),
          pltpu.VMEM_SHARED((NP, D), jnp.float32),
          pltpu.VMEM_SHARED((NP,), jnp.float32),
          pltpu.SemaphoreType.DMA,
          pltpu.SemaphoreType.DMA,
      ],
  )
  def c_kernel(row_hbm, vals_hbm, acc_out, cnt_out,
               ridx_v, valsb_v, ones_v, zcnt_v, acc_s, cnt_s, semA, semB):
    cid = lax.axis_index("c")
    sid = lax.axis_index("s")
    base = (cid * NS + sid) * epw
    sems = (semA, semB)

    zero16 = jnp.zeros((16,), jnp.float32)
    one16 = jnp.ones((16,), jnp.float32)

    def zfill(i, _):
      for k in range(D // 16):
        valsb_v[0, i, pl.ds(k * 16, 16)] = zero16
      return 0
    lax.fori_loop(0, C, zfill, 0)

    def zcnt_fill(i, _):
      zcnt_v[pl.ds(i * 16, 16)] = zero16
      return 0
    lax.fori_loop(0, ROWS_PER_TILE // 16, zcnt_fill, 0)

    for o in range(0, C, 16):
      ones_v[pl.ds(o, 16)] = one16

    for j in range(ROWS_PER_TILE // C):
      pltpu.sync_copy(valsb_v.at[0], acc_s.at[pl.ds(sid * ROWS_PER_TILE + j * C, C)])
    pltpu.sync_copy(zcnt_v, cnt_s.at[pl.ds(sid * ROWS_PER_TILE, ROWS_PER_TILE)])
    plsc.subcore_barrier()

    def fire(g, p):
      eb = base + g * C
      pltpu.sync_copy(row_hbm.at[pl.ds(off + eb, C)], ridx_v.at[p])
      pltpu.async_copy(vals_hbm.at[pl.ds(eb, C)], valsb_v.at[p], sems[p])

    def wait(p):
      pltpu.make_async_copy(vals_hbm.at[pl.ds(0, C)], valsb_v.at[p], sems[p]).wait()

    def scatter(p):
      pltpu.sync_copy(valsb_v.at[p], acc_s.at[ridx_v.at[p]], add=True)
      pltpu.sync_copy(ones_v, cnt_s.at[ridx_v.at[p]], add=True)

    fire(0, 0)

    def superstep(i, _):
      g0 = 2 * i
      fire(g0 + 1, 1)
      wait(0)
      scatter(0)
      fire(g0 + 2, 0)
      wait(1)
      scatter(1)
      return 0

    lax.fori_loop(0, nsuper, superstep, 0)
    if nchunk % 2:
      wait(0)
      scatter(0)
    else:
      fire(nchunk - 1, 1)
      wait(0)
      scatter(0)
      wait(1)
      scatter(1)
    plsc.subcore_barrier()

    pltpu.sync_copy(acc_s.at[pl.ds(sid * ROWS_PER_TILE, ROWS_PER_TILE)],
                    acc_out.at[cid, pl.ds(sid * ROWS_PER_TILE, ROWS_PER_TILE)])
    pltpu.sync_copy(cnt_s.at[pl.ds(sid * ROWS_PER_TILE, ROWS_PER_TILE)],
                    cnt_out.at[cid, pl.ds(sid * ROWS_PER_TILE, ROWS_PER_TILE)])

  return c_kernel(row, vals)


def _tc_finish_body(acc_ref, cnt_ref, acc2_ref, cnt2_ref, wl_ref, bl_ref, out_ref):
  feat = acc_ref[0] + acc_ref[1] + acc2_ref[0] + acc2_ref[1]
  cnt = cnt_ref[0] + cnt_ref[1] + cnt2_ref[0] + cnt2_ref[1]
  inv = 1.0 / jnp.maximum(cnt, 1.0)
  mean = feat * inv[:, None]
  h = lax.dot_general(mean, wl_ref[...], (((1,), (1,)), ((), ())),
                      preferred_element_type=jnp.float32)
  h = h + bl_ref[...][None, :]
  out_ref[...] = 0.5 * h * (1.0 + lax.erf(h * (2.0 ** -0.5)))


def _tc_finish(acc, cnt, acc2, cnt2, W_lin, b_lin):
  return pl.pallas_call(
      _tc_finish_body,
      out_shape=jax.ShapeDtypeStruct((NP, D), jnp.float32),
  )(acc, cnt, acc2, cnt2, W_lin, b_lin)


@jax.jit
def kernel(x, edge_index, curvature, W_lin, b_lin, W_curv, b_curv):
  row = edge_index[0]
  col = edge_index[1]
  wc = W_curv[:, 0]
  s0, s1 = SEGS
  xg0, d0 = _stage_a(row, col, curvature, x, s0["off"], s0["epw"], s0["nchunk"], s0["ep"])
  v0 = _stage_b(xg0, d0, wc, b_curv, s0["ep"])
  xg1, d1 = _stage_a(row, col, curvature, x, s1["off"], s1["epw"], s1["nchunk"], s1["ep"])
  v1 = _stage_b(xg1, d1, wc, b_curv, s1["ep"])
  acc0, cnt0 = _stage_c(row, v0, s0["off"], s0["epw"], s0["nchunk"])
  acc1, cnt1 = _stage_c(row, v1, s1["off"], s1["epw"], s1["nchunk"])
  return _tc_finish(acc0, cnt0, acc1, cnt1, W_lin, b_lin)[:N]


# speedup vs baseline: 1.3437x; 1.0345x over previous
import functools

import jax
import jax.numpy as jnp
from jax import lax
from jax.experimental import pallas as pl
from jax.experimental.pallas import tpu as pltpu
from jax.experimental.pallas import tpu_sc as plsc

N = 10000
E = 320000
D = 128

NC = 2
NS = 16
NW = NC * NS
C = 80
NP = 10240
ROWS_PER_TILE = NP // NS

SEGS = (
    dict(off=0, epw=6000, nchunk=75, ep=94 * 2048),
    dict(off=192000, epw=4000, nchunk=50, ep=63 * 2048),
)

_MESH = plsc.VectorSubcoreMesh(core_axis_name="c", subcore_axis_name="s")


def _stage_a(row, col, curv, x, off, epw, nchunk, ep):
  nsuper = (nchunk - 1) // 2 if nchunk % 2 else (nchunk - 2) // 2

  @functools.partial(
      pl.kernel,
      mesh=_MESH,
      out_type=[
          jax.ShapeDtypeStruct((ep, D), jnp.float32),
          jax.ShapeDtypeStruct((ep,), jnp.float32),
      ],
      scratch_types=[
          pltpu.VMEM((epw,), jnp.int32),
          pltpu.VMEM((epw,), jnp.int32),
          pltpu.VMEM((2, C, D), jnp.float32),
          pltpu.VMEM((2, C), jnp.float32),
          pltpu.VMEM((2, C), jnp.float32),
          pltpu.VMEM((epw,), jnp.float32),
          pltpu.SemaphoreType.DMA,
          pltpu.SemaphoreType.DMA,
      ],
  )
  def a_kernel(row_hbm, col_hbm, curv_hbm, x_hbm, xg_out, dif_out,
               ridx_v, cidx_v, rowsg_v, cr_v, cc_v, dif_v, semA, semB):
    cid = lax.axis_index("c")
    sid = lax.axis_index("s")
    base = (cid * NS + sid) * epw
    sems = (semA, semB)

    pltpu.sync_copy(row_hbm.at[pl.ds(off + base, epw)], ridx_v)
    pltpu.sync_copy(col_hbm.at[pl.ds(off + base, epw)], cidx_v)

    def fire(g, p):
      gb = g * C
      pltpu.async_copy(x_hbm.at[cidx_v.at[pl.ds(gb, C)]], rowsg_v.at[p], sems[p])
      pltpu.async_copy(curv_hbm.at[ridx_v.at[pl.ds(gb, C)]], cr_v.at[p], sems[p])
      pltpu.async_copy(curv_hbm.at[cidx_v.at[pl.ds(gb, C)]], cc_v.at[p], sems[p])

    def wait(p):
      pltpu.make_async_copy(x_hbm.at[cidx_v.at[pl.ds(0, C)]], rowsg_v.at[p], sems[p]).wait()
      pltpu.make_async_copy(curv_hbm.at[ridx_v.at[pl.ds(0, C)]], cr_v.at[p], sems[p]).wait()
      pltpu.make_async_copy(curv_hbm.at[ridx_v.at[pl.ds(0, C)]], cc_v.at[p], sems[p]).wait()

    def emit(g, p):
      gb = g * C
      for o in range(0, C, 16):
        dif_v[pl.ds(gb + o, 16)] = jnp.abs(cr_v[p, pl.ds(o, 16)] - cc_v[p, pl.ds(o, 16)])
      pltpu.sync_copy(rowsg_v.at[p], xg_out.at[pl.ds(base + gb, C)])

    fire(0, 0)

    def superstep(i, _):
      g0 = 2 * i
      fire(g0 + 1, 1)
      wait(0)
      emit(g0, 0)
      fire(g0 + 2, 0)
      wait(1)
      emit(g0 + 1, 1)
      return 0

    lax.fori_loop(0, nsuper, superstep, 0)
    if nchunk % 2:
      wait(0)
      emit(nchunk - 1, 0)
    else:
      fire(nchunk - 1, 1)
      wait(0)
      emit(nchunk - 2, 0)
      wait(1)
      emit(nchunk - 1, 1)
    pltpu.sync_copy(dif_v, dif_out.at[pl.ds(base, epw)])

  return a_kernel(row, col, curv, x)


def _gate_body(xg_ref, dif_ref, wc_ref, bc_ref, out_ref):
  z = dif_ref[...][:, None] * wc_ref[...] + bc_ref[...]
  w = 1.0 / (1.0 + jnp.exp(-z))
  out_ref[...] = xg_ref[...] * w


def _stage_b(xg, dif, wc, bc, ep):
  B = 2048
  grid = ep // B
  return pl.pallas_call(
      _gate_body,
      grid=(grid,),
      in_specs=[
          pl.BlockSpec((B, D), lambda i: (i, 0)),
          pl.BlockSpec((B,), lambda i: (i,)),
          pl.BlockSpec((1, D), lambda i: (0, 0)),
          pl.BlockSpec((1, D), lambda i: (0, 0)),
      ],
      out_specs=pl.BlockSpec((B, D), lambda i: (i, 0)),
      out_shape=jax.ShapeDtypeStruct((ep, D), jnp.float32),
  )(xg, dif, wc[None, :], bc[None, :])


def _stage_c(row, vals, off, epw, nchunk):
  nsuper = (nchunk - 1) // 2 if nchunk % 2 else (nchunk - 2) // 2

  @functools.partial(
      pl.kernel,
      mesh=_MESH,
      out_type=[
          jax.ShapeDtypeStruct((NC, NP, D), jnp.float32),
          jax.ShapeDtypeStruct((NC, NP), jnp.float32),
      ],
      scratch_types=[
          pltpu.VMEM((2, C), jnp.int32),
          pltpu.VMEM((2, C, D), jnp.float32),
          pltpu.VMEM((C,), jnp.float32),
          pltpu.VMEM((ROWS_PER_TILE,), jnp.float32),
          pltpu.VMEM_SHARED((NP, D), jnp.float32),
          pltpu.VMEM_SHARED((NP,), jnp.float32),
          pltpu.SemaphoreType.DMA,
          pltpu.SemaphoreType.DMA,
      ],
  )
  def c_kernel(row_hbm, vals_hbm, acc_out, cnt_out,
               ridx_v, valsb_v, ones_v, zcnt_v, acc_s, cnt_s, semA, semB):
    cid = lax.axis_index("c")
    sid = lax.axis_index("s")
    base = (cid * NS + sid) * epw
    sems = (semA, semB)

    zero16 = jnp.zeros((16,), jnp.float32)
    one16 = jnp.ones((16,), jnp.float32)

    def zfill(i, _):
      for k in range(D // 16):
        valsb_v[0, i, pl.ds(k * 16, 16)] = zero16
      return 0
    lax.fori_loop(0, C, zfill, 0)

    def zcnt_fill(i, _):
      zcnt_v[pl.ds(i * 16, 16)] = zero16
      return 0
    lax.fori_loop(0, ROWS_PER_TILE // 16, zcnt_fill, 0)

    for o in range(0, C, 16):
      ones_v[pl.ds(o, 16)] = one16

    for j in range(ROWS_PER_TILE // C):
      pltpu.sync_copy(valsb_v.at[0], acc_s.at[pl.ds(sid * ROWS_PER_TILE + j * C, C)])
    pltpu.sync_copy(zcnt_v, cnt_s.at[pl.ds(sid * ROWS_PER_TILE, ROWS_PER_TILE)])
    plsc.subcore_barrier()

    def fire(g, p):
      eb = base + g * C
      pltpu.sync_copy(row_hbm.at[pl.ds(off + eb, C)], ridx_v.at[p])
      pltpu.async_copy(vals_hbm.at[pl.ds(eb, C)], valsb_v.at[p], sems[p])

    def wait(p):
      pltpu.make_async_copy(vals_hbm.at[pl.ds(0, C)], valsb_v.at[p], sems[p]).wait()

    def scatter(p):
      pltpu.sync_copy(valsb_v.at[p], acc_s.at[ridx_v.at[p]], add=True)
      pltpu.sync_copy(ones_v, cnt_s.at[ridx_v.at[p]], add=True)

    fire(0, 0)

    def superstep(i, _):
      g0 = 2 * i
      fire(g0 + 1, 1)
      wait(0)
      scatter(0)
      fire(g0 + 2, 0)
      wait(1)
      scatter(1)
      return 0

    lax.fori_loop(0, nsuper, superstep, 0)
    if nchunk % 2:
      wait(0)
      scatter(0)
    else:
      fire(nchunk - 1, 1)
      wait(0)
      scatter(0)
      wait(1)
      scatter(1)
    plsc.subcore_barrier()

    pltpu.sync_copy(acc_s.at[pl.ds(sid * ROWS_PER_TILE, ROWS_PER_TILE)],
                    acc_out.at[cid, pl.ds(sid * ROWS_PER_TILE, ROWS_PER_TILE)])
    pltpu.sync_copy(cnt_s.at[pl.ds(sid * ROWS_PER_TILE, ROWS_PER_TILE)],
                    cnt_out.at[cid, pl.ds(sid * ROWS_PER_TILE, ROWS_PER_TILE)])

  return c_kernel(row, vals)


def _tc_finish_body(acc_ref, cnt_ref, acc2_ref, cnt2_ref, wl_ref, bl_ref, out_ref):
  feat = acc_ref[0] + acc_ref[1] + acc2_ref[0] + acc2_ref[1]
  cnt = cnt_ref[0] + cnt_ref[1] + cnt2_ref[0] + cnt2_ref[1]
  inv = 1.0 / jnp.maximum(cnt, 1.0)
  mean = feat * inv[:, None]
  h = lax.dot_general(mean, wl_ref[...], (((1,), (1,)), ((), ())),
                      preferred_element_type=jnp.float32)
  h = h + bl_ref[...][None, :]
  out_ref[...] = 0.5 * h * (1.0 + lax.erf(h * (2.0 ** -0.5)))


def _tc_finish(acc, cnt, acc2, cnt2, W_lin, b_lin):
  return pl.pallas_call(
      _tc_finish_body,
      out_shape=jax.ShapeDtypeStruct((NP, D), jnp.float32),
  )(acc, cnt, acc2, cnt2, W_lin, b_lin)


@jax.jit
def kernel(x, edge_index, curvature, W_lin, b_lin, W_curv, b_curv):
  row = edge_index[0]
  col = edge_index[1]
  wc = W_curv[:, 0]
  s0, s1 = SEGS
  xg0, d0 = _stage_a(row, col, curvature, x, s0["off"], s0["epw"], s0["nchunk"], s0["ep"])
  v0 = _stage_b(xg0, d0, wc, b_curv, s0["ep"])
  xg1, d1 = _stage_a(row, col, curvature, x, s1["off"], s1["epw"], s1["nchunk"], s1["ep"])
  v1 = _stage_b(xg1, d1, wc, b_curv, s1["ep"])
  acc0, cnt0 = _stage_c(row, v0, s0["off"], s0["epw"], s0["nchunk"])
  acc1, cnt1 = _stage_c(row, v1, s1["off"], s1["epw"], s1["nchunk"])
  return _tc_finish(acc0, cnt0, acc1, cnt1, W_lin, b_lin)[:N]
